# baseline (speedup 1.0000x reference)
import functools

import jax
import jax.numpy as jnp
from jax import lax
from jax.experimental import pallas as pl
from jax.experimental.pallas import tpu as pltpu
from jax.experimental.pallas import tpu_sc as plsc

_NUM_CORES = 2
_NUM_SUBCORES = 16
_NUM_WORKERS = _NUM_CORES * _NUM_SUBCORES
_NBUF = 2


@functools.lru_cache(maxsize=None)
def _make_gather(batch, hist, vocab, dim, nr):
    rows_per_w = batch // _NUM_WORKERS
    n_chunks = rows_per_w // nr
    n_groups = n_chunks // _NBUF
    mesh = plsc.VectorSubcoreMesh(core_axis_name="c", subcore_axis_name="s")

    scratch = (
        [pltpu.VMEM((nr, hist), jnp.int32) for _ in range(_NBUF)]
        + [pltpu.VMEM((nr, hist, dim), jnp.float32) for _ in range(_NBUF)]
        + [pltpu.SemaphoreType.DMA for _ in range(3 * _NBUF)]
    )

    @functools.partial(
        pl.kernel,
        mesh=mesh,
        out_type=jax.ShapeDtypeStruct((batch, hist, dim), jnp.float32),
        scratch_types=scratch,
        compiler_params=pltpu.CompilerParams(use_tc_tiling_on_sc=False),
    )
    def gather_kernel(idx_hbm, table_hbm, out_hbm, *bufs):
        idx_bufs = bufs[0:_NBUF]
        row_bufs = bufs[_NBUF : 2 * _NBUF]
        idx_sems = bufs[2 * _NBUF : 3 * _NBUF]
        g_sems = bufs[3 * _NBUF : 4 * _NBUF]
        out_sems = bufs[4 * _NBUF : 5 * _NBUF]

        wid = lax.axis_index("s") * _NUM_CORES + lax.axis_index("c")
        base = wid * rows_per_w

        def start_idx(ci, b):
            pltpu.async_copy(
                idx_hbm.at[pl.ds(base + ci * nr, nr)], idx_bufs[b], idx_sems[b]
            )

        def wait_idx(b):
            pltpu.make_async_copy(
                idx_hbm.at[pl.ds(0, nr)], idx_bufs[b], idx_sems[b]
            ).wait()

        def start_gathers(b):
            for i in range(nr):
                pltpu.async_copy(
                    table_hbm.at[idx_bufs[b].at[i]], row_bufs[b].at[i], g_sems[b]
                )

        def wait_gathers(b):
            pltpu.make_async_copy(
                out_hbm.at[pl.ds(0, nr)], row_bufs[b], g_sems[b]
            ).wait()

        def start_out(ci, b):
            pltpu.async_copy(
                row_bufs[b], out_hbm.at[pl.ds(base + ci * nr, nr)], out_sems[b]
            )

        def wait_out(b):
            pltpu.make_async_copy(
                row_bufs[b], out_hbm.at[pl.ds(0, nr)], out_sems[b]
            ).wait()

        for b in range(_NBUF):
            start_idx(b, b)

        def outer(g, carry):
            for b in range(_NBUF):
                ci = g * _NBUF + b
                @pl.when(g > 0)
                def _():
                    wait_out(b)

                wait_idx(b)
                start_gathers(b)
                wait_gathers(b)
                start_out(ci, b)

                @pl.when(g < n_groups - 1)
                def _():
                    start_idx(ci + _NBUF, b)

            return carry

        lax.fori_loop(0, n_groups, outer, 0, unroll=False)

        for b in range(_NBUF):
            wait_out(b)

    return gather_kernel


def kernel(input_ids, table):
    batch, hist = input_ids.shape
    vocab, dim = table.shape
    ids = input_ids.astype(jnp.int32)
    return _make_gather(batch, hist, vocab, dim, 4)(ids, table)


# speedup vs baseline: 1.6572x; 1.6572x over previous
import functools

import jax
import jax.numpy as jnp
from jax import lax
from jax.experimental import pallas as pl
from jax.experimental.pallas import tpu as pltpu
from jax.experimental.pallas import tpu_sc as plsc

_NUM_CORES = 2
_NUM_SUBCORES = 16
_NUM_WORKERS = _NUM_CORES * _NUM_SUBCORES
_NBUF = 2


@functools.lru_cache(maxsize=None)
def _make_gather(batch, hist, vocab, dim, nr):
    rows_per_w = batch // _NUM_WORKERS
    n_chunks = rows_per_w // nr
    n_groups = n_chunks // _NBUF
    mesh = plsc.VectorSubcoreMesh(core_axis_name="c", subcore_axis_name="s")

    scratch = (
        [pltpu.VMEM((nr, hist), jnp.int32) for _ in range(_NBUF)]
        + [pltpu.VMEM((nr, hist, dim), jnp.float32) for _ in range(_NBUF)]
        + [pltpu.SemaphoreType.DMA for _ in range(3 * _NBUF)]
    )

    @functools.partial(
        pl.kernel,
        mesh=mesh,
        out_type=jax.ShapeDtypeStruct((batch, hist, 2 * dim), jnp.float32),
        scratch_types=scratch,
        compiler_params=pltpu.CompilerParams(use_tc_tiling_on_sc=False),
    )
    def gather_kernel(idx_hbm, table_hbm, out_hbm, *bufs):
        idx_bufs = bufs[0:_NBUF]
        row_bufs = bufs[_NBUF : 2 * _NBUF]
        idx_sems = bufs[2 * _NBUF : 3 * _NBUF]
        g_sems = bufs[3 * _NBUF : 4 * _NBUF]
        out_sems = bufs[4 * _NBUF : 5 * _NBUF]

        wid = lax.axis_index("s") * _NUM_CORES + lax.axis_index("c")
        base = wid * rows_per_w

        def start_idx(ci, b):
            pltpu.async_copy(
                idx_hbm.at[pl.ds(base + ci * nr, nr)], idx_bufs[b], idx_sems[b]
            )

        def wait_idx(b):
            pltpu.make_async_copy(
                idx_hbm.at[pl.ds(0, nr)], idx_bufs[b], idx_sems[b]
            ).wait()

        def start_gathers(b):
            for i in range(nr):
                pltpu.async_copy(
                    table_hbm.at[idx_bufs[b].at[i]], row_bufs[b].at[i], g_sems[b]
                )

        def wait_gathers(b):
            pltpu.make_async_copy(
                out_hbm.at[pl.ds(0, nr)], row_bufs[b], g_sems[b]
            ).wait()

        def start_out(ci, b):
            pltpu.async_copy(
                row_bufs[b],
                out_hbm.at[pl.ds(base + ci * nr, nr), :, pl.ds(0, dim)],
                out_sems[b],
            )

        def wait_out(b):
            pltpu.make_async_copy(
                row_bufs[b], out_hbm.at[pl.ds(0, nr), :, pl.ds(0, dim)], out_sems[b]
            ).wait()

        for b in range(_NBUF):
            start_idx(b, b)

        def outer(g, carry):
            for b in range(_NBUF):
                ci = g * _NBUF + b
                @pl.when(g > 0)
                def _():
                    wait_out(b)

                wait_idx(b)
                start_gathers(b)
                wait_gathers(b)
                start_out(ci, b)

                @pl.when(g < n_groups - 1)
                def _():
                    start_idx(ci + _NBUF, b)

            return carry

        lax.fori_loop(0, n_groups, outer, 0, unroll=False)

        for b in range(_NBUF):
            wait_out(b)

    return gather_kernel


def kernel(input_ids, table):
    batch, hist = input_ids.shape
    vocab, dim = table.shape
    ids = input_ids.astype(jnp.int32)
    out_wide = _make_gather(batch, hist, vocab, dim, 4)(ids, table)
    return out_wide[:, :, :dim]
